# baseline (speedup 1.0000x reference)
import functools

import jax
import jax.numpy as jnp
from jax import lax
from jax.experimental import pallas as pl
from jax.experimental.pallas import tpu as pltpu
from jax.experimental.pallas import tpu_sc as plsc

_INFO = plsc.get_sparse_core_info()
_NC, _NS, _L = _INFO.num_cores, _INFO.num_subcores, _INFO.num_lanes
_NW = _NC * _NS


def _make_sc_kernel(seq, dim, batch):
    bt_n = batch // 128
    dt_n = dim // 8
    n_units = seq * bt_n
    upw = n_units // _NW
    pairs = upw // 2
    mesh = plsc.VectorSubcoreMesh(core_axis_name="c", subcore_axis_name="s")

    @functools.partial(
        pl.kernel,
        mesh=mesh,
        compiler_params=pltpu.CompilerParams(
            use_tc_tiling_on_sc=False, needs_layout_passes=False
        ),
        out_type=jax.ShapeDtypeStruct((seq, dt_n, bt_n, 8, 128), jnp.float32),
        scratch_types=[
            pltpu.VMEM((upw, 128), jnp.int32),
            pltpu.VMEM((seq, dim), jnp.float32),
            pltpu.VMEM((128, dim), jnp.float32),
            pltpu.VMEM((128, dim), jnp.float32),
            pltpu.VMEM((128, dim), jnp.float32),
            pltpu.VMEM((dim, 128), jnp.float32),
            pltpu.VMEM((dim, 128), jnp.float32),
            pltpu.VMEM((dim, 128), jnp.float32),
            pltpu.SemaphoreType.DMA,
            pltpu.SemaphoreType.DMA,
            pltpu.SemaphoreType.DMA,
            pltpu.SemaphoreType.DMA,
            pltpu.SemaphoreType.DMA,
            pltpu.SemaphoreType.DMA,
        ],
    )
    def k(idx_hbm, pe_hbm, table_hbm, out_hbm,
          idx_v, pe_v, r0, r1, r2, t0, t1, t2, gs0, gs1, gs2, ss0, ss1, ss2):
        wid = lax.axis_index("s") * _NC + lax.axis_index("c")
        ubase = wid * upw
        pltpu.sync_copy(idx_hbm.at[pl.ds(ubase, upw)], idx_v)
        pltpu.sync_copy(pe_hbm, pe_v)

        lane = jax.lax.broadcasted_iota(jnp.int32, (_L,), 0)

        def gather_start(u, buf, sem):
            pltpu.async_copy(table_hbm.at[idx_v.at[u]], buf, sem)

        def gather_wait(buf, sem):
            pltpu.make_async_copy(table_hbm.at[idx_v.at[0]], buf, sem).wait()

        def store_wait(tbuf, sem):
            for _ in range(dt_n):
                pltpu.make_async_copy(
                    tbuf.at[pl.ds(0, 8)], out_hbm.at[0, 0, 0], sem
                ).wait()

        def transpose_add(rows, tbuf, s):
            pe_j = [pe_v[s, pl.ds(j * _L, _L)] for j in range(dim // _L)]
            d_j = [lane + j * _L for j in range(dim // _L)]

            @plsc.parallel_loop(0, 128, unroll=8)
            def _(b):
                b_idx = jnp.full((_L,), b, jnp.int32)
                for j in range(dim // _L):
                    val = rows[b, pl.ds(j * _L, _L)] + pe_j[j]
                    plsc.store_scatter(tbuf, [d_j[j], b_idx], val)

        def unit_coords(u):
            r = ubase + u
            s_t = r // (bt_n * 8)
            rem = lax.rem(r, bt_n * 8)
            b_t = rem // 8
            s = s_t * 8 + lax.rem(rem, 8)
            return s, b_t

        def store_start(tbuf, s, b_t, sem):
            for dt in range(dt_n):
                pltpu.async_copy(
                    tbuf.at[pl.ds(dt * 8, 8)], out_hbm.at[s, dt, b_t], sem
                )

        gather_start(0, r0, gs0)
        gather_start(1, r1, gs1)
        gather_start(2, r2, gs2)
        n_slots = (upw + 2) // 3

        def slot_body(i, _):
            for k_, (rb, tb, gs, ss) in enumerate(
                ((r0, t0, gs0, ss0), (r1, t1, gs1, ss1), (r2, t2, gs2, ss2))):
                u = 3 * i + k_

                @pl.when(u < upw)
                def _():
                    gather_wait(rb, gs)

                    @pl.when(u >= 3)
                    def _():
                        store_wait(tb, ss)

                    s, b_t = unit_coords(u)
                    transpose_add(rb, tb, s)

                    @pl.when(u + 3 < upw)
                    def _():
                        gather_start(u + 3, rb, gs)

                    store_start(tb, s, b_t, ss)
            return 0

        lax.fori_loop(0, n_slots, slot_body, 0)
        store_wait(t0, ss0)
        store_wait(t1, ss1)
        store_wait(t2, ss2)

    return k


def kernel(x, table):
    b, s = x.shape
    v, d = table.shape

    pos = jnp.arange(s, dtype=jnp.float32)
    pe = jnp.tile(jnp.stack([jnp.sin(pos), jnp.cos(pos)], axis=1), (1, d // 2))

    x4 = (
        x.T.reshape(s // 8, 8, b // 128, 128)
        .transpose(0, 2, 1, 3)
        .reshape(s * b // 128, 128)
        .astype(jnp.int32)
    )

    out5 = _make_sc_kernel(s, d, b)(x4, pe, table)
    return out5.transpose(2, 4, 0, 1, 3).reshape(b, s, d)


# speedup vs baseline: 1.0247x; 1.0247x over previous
import functools

import jax
import jax.numpy as jnp
from jax import lax
from jax.experimental import pallas as pl
from jax.experimental.pallas import tpu as pltpu
from jax.experimental.pallas import tpu_sc as plsc

_INFO = plsc.get_sparse_core_info()
_NC, _NS, _L = _INFO.num_cores, _INFO.num_subcores, _INFO.num_lanes
_NW = _NC * _NS


def _make_sc_kernel(n_rows, chunk, seq, dim):
    n_chunks_total = n_rows // chunk
    chunks_per_w = n_chunks_total // _NW
    pe_steps = seq // chunk
    mesh = plsc.VectorSubcoreMesh(core_axis_name="c", subcore_axis_name="s")

    half = chunks_per_w // 2

    @functools.partial(
        pl.kernel,
        mesh=mesh,
        compiler_params=pltpu.CompilerParams(use_tc_tiling_on_sc=False),
        out_type=jax.ShapeDtypeStruct((n_chunks_total, chunk, dim), jnp.float32),
        scratch_types=[
            pltpu.VMEM((chunks_per_w, chunk), jnp.int32),
            pltpu.VMEM((seq, dim), jnp.float32),
            pltpu.VMEM((chunk, dim), jnp.float32),
            pltpu.VMEM((chunk, dim), jnp.float32),
            pltpu.VMEM((chunk, dim), jnp.float32),
            pltpu.VMEM((chunk, dim), jnp.float32),
            pltpu.SemaphoreType.DMA,
            pltpu.SemaphoreType.DMA,
            pltpu.SemaphoreType.DMA,
            pltpu.SemaphoreType.DMA,
        ],
    )
    def k(idx_hbm, pe_hbm, table_hbm, out_hbm,
          idx_v, pe_v, g0, g1, s0, s1, gs0, gs1, ss0, ss1):
        wid = lax.axis_index("s") * _NC + lax.axis_index("c")
        cbase = wid * chunks_per_w
        pltpu.sync_copy(idx_hbm.at[pl.ds(cbase, chunks_per_w)], idx_v)
        pltpu.sync_copy(pe_hbm, pe_v)

        def gather_start(c, buf, sem):
            pltpu.async_copy(table_hbm.at[idx_v.at[c]], buf, sem)

        def gather_wait(buf, sem):
            pltpu.make_async_copy(table_hbm.at[idx_v.at[0]], buf, sem).wait()

        def store_wait(buf, sem):
            pltpu.make_async_copy(buf, out_hbm.at[cbase], sem).wait()

        def add(src, dst, po):
            @plsc.parallel_loop(0, chunk, unroll=4)
            def _(r):
                for j in range(dim // _L):
                    s = pl.ds(j * _L, _L)
                    dst[r, s] = src[r, s] + pe_v[po + r, s]

        gather_start(0, g0, gs0)
        gather_start(1, g1, gs1)

        def pair_body(i, _):
            for b, (g, s, gs, ss) in enumerate(
                ((g0, s0, gs0, ss0), (g1, s1, gs1, ss1))):
                c = 2 * i + b
                gather_wait(g, gs)

                @pl.when(i >= 1)
                def _():
                    store_wait(s, ss)

                add(g, s, lax.rem(c, pe_steps) * chunk)

                @pl.when(i < half - 1)
                def _():
                    gather_start(c + 2, g, gs)

                pltpu.async_copy(s, out_hbm.at[cbase + c], ss)
            return 0

        lax.fori_loop(0, half, pair_body, 0)
        store_wait(s0, ss0)
        store_wait(s1, ss1)

    return k


def kernel(x, table):
    b, s = x.shape
    v, d = table.shape
    n_rows = b * s
    chunk = 100

    pos = jnp.arange(s, dtype=jnp.float32)
    pe = jnp.tile(jnp.stack([jnp.sin(pos), jnp.cos(pos)], axis=1), (1, d // 2))

    idx = x.reshape(n_rows // chunk, chunk).astype(jnp.int32)
    out = _make_sc_kernel(n_rows, chunk, s, d)(idx, pe, table)
    return out.reshape(b, s, d)


# speedup vs baseline: 1.0249x; 1.0002x over previous
import functools

import jax
import jax.numpy as jnp
from jax import lax
from jax.experimental import pallas as pl
from jax.experimental.pallas import tpu as pltpu
from jax.experimental.pallas import tpu_sc as plsc

_INFO = plsc.get_sparse_core_info()
_NC, _NS, _L = _INFO.num_cores, _INFO.num_subcores, _INFO.num_lanes
_NW = _NC * _NS


def _make_sc_kernel(n_rows, chunk, seq, dim):
    n_chunks_total = n_rows // chunk
    chunks_per_w = n_chunks_total // _NW
    pe_steps = seq // chunk
    mesh = plsc.VectorSubcoreMesh(core_axis_name="c", subcore_axis_name="s")

    half = chunks_per_w // 2

    @functools.partial(
        pl.kernel,
        mesh=mesh,
        compiler_params=pltpu.CompilerParams(use_tc_tiling_on_sc=False),
        out_type=jax.ShapeDtypeStruct((n_chunks_total, chunk, dim), jnp.float32),
        scratch_types=[
            pltpu.VMEM((chunks_per_w, chunk), jnp.int32),
            pltpu.VMEM((seq, dim), jnp.float32),
            pltpu.VMEM((chunk, dim), jnp.float32),
            pltpu.VMEM((chunk, dim), jnp.float32),
            pltpu.VMEM((chunk, dim), jnp.float32),
            pltpu.VMEM((chunk, dim), jnp.float32),
            pltpu.SemaphoreType.DMA,
            pltpu.SemaphoreType.DMA,
            pltpu.SemaphoreType.DMA,
            pltpu.SemaphoreType.DMA,
        ],
    )
    def k(idx_hbm, pe_hbm, table_hbm, out_hbm,
          idx_v, pe_v, g0, g1, s0, s1, gs0, gs1, ss0, ss1):
        wid = lax.axis_index("s") * _NC + lax.axis_index("c")
        cbase = wid * chunks_per_w
        pltpu.sync_copy(idx_hbm.at[pl.ds(cbase, chunks_per_w)], idx_v)
        pltpu.sync_copy(pe_hbm, pe_v)

        def gather_start(c, buf, sem):
            pltpu.async_copy(table_hbm.at[idx_v.at[c]], buf, sem)

        def gather_wait(buf, sem):
            pltpu.make_async_copy(table_hbm.at[idx_v.at[0]], buf, sem).wait()

        def store_wait(buf, sem):
            pltpu.make_async_copy(buf, out_hbm.at[cbase], sem).wait()

        def add(src, dst, po):
            @plsc.parallel_loop(0, chunk, unroll=10)
            def _(r):
                for j in range(dim // _L):
                    s = pl.ds(j * _L, _L)
                    dst[r, s] = src[r, s] + pe_v[po + r, s]

        gather_start(0, g0, gs0)
        gather_start(1, g1, gs1)

        def pair_body(i, _):
            for b, (g, s, gs, ss) in enumerate(
                ((g0, s0, gs0, ss0), (g1, s1, gs1, ss1))):
                c = 2 * i + b
                gather_wait(g, gs)

                @pl.when(i >= 1)
                def _():
                    store_wait(s, ss)

                add(g, s, lax.rem(c, pe_steps) * chunk)

                @pl.when(i < half - 1)
                def _():
                    gather_start(c + 2, g, gs)

                pltpu.async_copy(s, out_hbm.at[cbase + c], ss)
            return 0

        lax.fori_loop(0, half, pair_body, 0)
        store_wait(s0, ss0)
        store_wait(s1, ss1)

    return k


def kernel(x, table):
    b, s = x.shape
    v, d = table.shape
    n_rows = b * s
    chunk = 100

    pos = jnp.arange(s, dtype=jnp.float32)
    pe = jnp.tile(jnp.stack([jnp.sin(pos), jnp.cos(pos)], axis=1), (1, d // 2))

    idx = x.reshape(n_rows // chunk, chunk).astype(jnp.int32)
    out = _make_sc_kernel(n_rows, chunk, s, d)(idx, pe, table)
    return out.reshape(b, s, d)
